# baseline (speedup 1.0000x reference)
import functools
import jax
import jax.numpy as jnp
from jax import lax
from jax.experimental import pallas as pl
from jax.experimental.pallas import tpu as pltpu
from jax.experimental.pallas import tpu_sc as plsc

NUM_ENT = 100000
NUM_REL = 1000
ENT_DIM = 90
TIME_DIM = 10
BATCH = 16384
MEGA_W = 256
REL_W = 128

_NC, _NS = 2, 16
_NW = _NC * _NS
_BPW = BATCH // _NW
_CHUNK = 128
_NCHUNK = _BPW // _CHUNK


def _sc_gather_body(heads_hbm, tails_hbm, rels_hbm, mega_hbm, rel_hbm,
                    h_out, t_out, r_out,
                    hidx, tidx, ridx, hrows, trows, rrows, sem):
    wid = lax.axis_index("s") * _NC + lax.axis_index("c")
    for ci in range(_NCHUNK):
        base = wid * _BPW + ci * _CHUNK
        pltpu.sync_copy(heads_hbm.at[pl.ds(base, _CHUNK)], hidx)
        pltpu.sync_copy(tails_hbm.at[pl.ds(base, _CHUNK)], tidx)
        pltpu.sync_copy(rels_hbm.at[pl.ds(base, _CHUNK)], ridx)
        d1 = pltpu.async_copy(mega_hbm.at[hidx], hrows, sem)
        d2 = pltpu.async_copy(mega_hbm.at[tidx], trows, sem)
        d3 = pltpu.async_copy(rel_hbm.at[ridx], rrows, sem)
        d1.wait()
        d2.wait()
        d3.wait()
        pltpu.sync_copy(hrows, h_out.at[pl.ds(base, _CHUNK)])
        pltpu.sync_copy(trows, t_out.at[pl.ds(base, _CHUNK)])
        pltpu.sync_copy(rrows, r_out.at[pl.ds(base, _CHUNK)])


_sc_gather_cache = []


def _get_sc_gather():
    if not _sc_gather_cache:
        _sc_gather_cache.append(_build_sc_gather())
    return _sc_gather_cache[0]


def _build_sc_gather():
    return pl.kernel(
        _sc_gather_body,
        out_type=(
            jax.ShapeDtypeStruct((BATCH, MEGA_W), jnp.float32),
            jax.ShapeDtypeStruct((BATCH, MEGA_W), jnp.float32),
            jax.ShapeDtypeStruct((BATCH, REL_W), jnp.float32),
        ),
        mesh=plsc.VectorSubcoreMesh(core_axis_name="c", subcore_axis_name="s",
                                    num_cores=_NC, num_subcores=_NS),
        compiler_params=pltpu.CompilerParams(),
        scratch_types=[
            pltpu.VMEM((_CHUNK,), jnp.int32),
            pltpu.VMEM((_CHUNK,), jnp.int32),
            pltpu.VMEM((_CHUNK,), jnp.int32),
            pltpu.VMEM((_CHUNK, MEGA_W), jnp.float32),
            pltpu.VMEM((_CHUNK, MEGA_W), jnp.float32),
            pltpu.VMEM((_CHUNK, REL_W), jnp.float32),
            pltpu.SemaphoreType.DMA,
        ],
    )


_BLK = 512


def _tc_score_body(h_ref, t_ref, r_ref, y_ref, mo_ref, dy_ref, out_ref):
    y = y_ref[...]
    mo = mo_ref[...]
    dy = dy_ref[...]
    tvec = jnp.concatenate(
        [jnp.broadcast_to(y, (_BLK, TIME_DIM)),
         jnp.broadcast_to(mo, (_BLK, TIME_DIM)),
         jnp.broadcast_to(dy, (_BLK, TIME_DIM)),
         jnp.zeros((_BLK, 2), jnp.float32)], axis=1)

    h = h_ref[...]
    t = t_ref[...]
    r = r_ref[...]
    henc = h[:, 160:192] * jnp.sin(h[:, 96:128] * tvec + h[:, 128:160])
    tenc = t[:, 160:192] * jnp.sin(t[:, 96:128] * tvec + t[:, 128:160])
    denc = henc - tenc
    d10 = (denc[:, 0:10] + denc[:, 10:20] + denc[:, 20:30] + r[:, 96:106])
    d96 = h[:, :96] + r[:, :96] - t[:, :96]
    s = jnp.sum(d96 * d96, axis=1) + jnp.sum(d10 * d10, axis=1)
    out_ref[...] = jnp.sqrt(s)[:, None]


def _tc_score(h, t, r, y2, m2, d2, interpret=False):
    return pl.pallas_call(
        _tc_score_body,
        grid=(BATCH // _BLK,),
        in_specs=[
            pl.BlockSpec((_BLK, MEGA_W), lambda i: (i, 0)),
            pl.BlockSpec((_BLK, MEGA_W), lambda i: (i, 0)),
            pl.BlockSpec((_BLK, REL_W), lambda i: (i, 0)),
            pl.BlockSpec((_BLK, 1), lambda i: (i, 0)),
            pl.BlockSpec((_BLK, 1), lambda i: (i, 0)),
            pl.BlockSpec((_BLK, 1), lambda i: (i, 0)),
        ],
        out_specs=pl.BlockSpec((_BLK, 1), lambda i: (i, 0)),
        out_shape=jax.ShapeDtypeStruct((BATCH, 1), jnp.float32),
        interpret=interpret,
    )(h, t, r, y2, m2, d2)


def _pack_tables(entity_emb, relation_emb, year_freq, month_freq, day_freq,
                 year_phi, month_phi, day_phi, year_amp, month_amp, day_amp):
    zn = lambda w: jnp.zeros((NUM_ENT, w), jnp.float32)
    mega = jnp.concatenate(
        [entity_emb, zn(6),
         year_freq, month_freq, day_freq, zn(2),
         year_phi, month_phi, day_phi, zn(2),
         year_amp, month_amp, day_amp, zn(66)], axis=1)
    zr = lambda w: jnp.zeros((NUM_REL, w), jnp.float32)
    relpad = jnp.concatenate(
        [relation_emb[:, :ENT_DIM], zr(6), relation_emb[:, ENT_DIM:], zr(22)],
        axis=1)
    return mega, relpad


def kernel(heads, rels, tails, years, months, days, entity_emb, relation_emb,
           year_freq, month_freq, day_freq, year_phi, month_phi, day_phi,
           year_amp, month_amp, day_amp):
    mega, relpad = _pack_tables(entity_emb, relation_emb, year_freq,
                                month_freq, day_freq, year_phi, month_phi,
                                day_phi, year_amp, month_amp, day_amp)
    hi = heads.astype(jnp.int32)
    ti = tails.astype(jnp.int32)
    ri = rels.astype(jnp.int32)
    h, t, r = _get_sc_gather()(hi, ti, ri, mega, relpad)
    y2 = years.reshape(BATCH, 1)
    m2 = months.reshape(BATCH, 1)
    d2 = days.reshape(BATCH, 1)
    scores = _tc_score(h, t, r, y2, m2, d2)
    return scores.reshape(-1)


# speedup vs baseline: 1.4522x; 1.4522x over previous
import functools
import jax
import jax.numpy as jnp
from jax import lax
from jax.experimental import pallas as pl
from jax.experimental.pallas import tpu as pltpu
from jax.experimental.pallas import tpu_sc as plsc

NUM_ENT = 100000
NUM_REL = 1000
ENT_DIM = 90
TIME_DIM = 10
BATCH = 16384
MEGA_W = 256
REL_W = 128

_NC, _NS = 2, 16
_NW = _NC * _NS
_BPW = BATCH // _NW
_CHUNK = 64
_NCHUNK = _BPW // _CHUNK


def _sc_gather_body(heads_hbm, tails_hbm, rels_hbm, mega_hbm, rel_hbm,
                    h_out, t_out, r_out,
                    hidx, tidx, ridx,
                    hb0, hb1, tb0, tb1, rb0, rb1,
                    gsem, wsem0, wsem1):
    wid = lax.axis_index("s") * _NC + lax.axis_index("c")
    base = wid * _BPW
    pltpu.sync_copy(heads_hbm.at[pl.ds(base, _BPW)], hidx)
    pltpu.sync_copy(tails_hbm.at[pl.ds(base, _BPW)], tidx)
    pltpu.sync_copy(rels_hbm.at[pl.ds(base, _BPW)], ridx)

    hb = (hb0, hb1)
    tb = (tb0, tb1)
    rb = (rb0, rb1)
    wsem = (wsem0, wsem1)
    gd = [None] * _NCHUNK
    wd = [None] * _NCHUNK

    def issue_gather(ci):
        b = ci % 2
        s = pl.ds(ci * _CHUNK, _CHUNK)
        gd[ci] = (
            pltpu.async_copy(mega_hbm.at[hidx.at[s]], hb[b], gsem),
            pltpu.async_copy(mega_hbm.at[tidx.at[s]], tb[b], gsem),
            pltpu.async_copy(rel_hbm.at[ridx.at[s]], rb[b], gsem),
        )

    def issue_write(ci):
        b = ci % 2
        o = pl.ds(base + ci * _CHUNK, _CHUNK)
        wd[ci] = (
            pltpu.async_copy(hb[b], h_out.at[o], wsem[b]),
            pltpu.async_copy(tb[b], t_out.at[o], wsem[b]),
            pltpu.async_copy(rb[b], r_out.at[o], wsem[b]),
        )

    issue_gather(0)
    for ci in range(_NCHUNK):
        for d in gd[ci]:
            d.wait()
        issue_write(ci)
        if ci + 1 < _NCHUNK:
            if ci >= 1:
                for d in wd[ci - 1]:
                    d.wait()
            issue_gather(ci + 1)
    for d in wd[_NCHUNK - 2]:
        d.wait()
    for d in wd[_NCHUNK - 1]:
        d.wait()


_sc_gather_cache = []


def _get_sc_gather():
    if not _sc_gather_cache:
        _sc_gather_cache.append(_build_sc_gather())
    return _sc_gather_cache[0]


def _build_sc_gather():
    return pl.kernel(
        _sc_gather_body,
        out_type=(
            jax.ShapeDtypeStruct((BATCH, MEGA_W), jnp.float32),
            jax.ShapeDtypeStruct((BATCH, MEGA_W), jnp.float32),
            jax.ShapeDtypeStruct((BATCH, REL_W), jnp.float32),
        ),
        mesh=plsc.VectorSubcoreMesh(core_axis_name="c", subcore_axis_name="s",
                                    num_cores=_NC, num_subcores=_NS),
        compiler_params=pltpu.CompilerParams(),
        scratch_types=[
            pltpu.VMEM((_BPW,), jnp.int32),
            pltpu.VMEM((_BPW,), jnp.int32),
            pltpu.VMEM((_BPW,), jnp.int32),
            pltpu.VMEM((_CHUNK, MEGA_W), jnp.float32),
            pltpu.VMEM((_CHUNK, MEGA_W), jnp.float32),
            pltpu.VMEM((_CHUNK, MEGA_W), jnp.float32),
            pltpu.VMEM((_CHUNK, MEGA_W), jnp.float32),
            pltpu.VMEM((_CHUNK, REL_W), jnp.float32),
            pltpu.VMEM((_CHUNK, REL_W), jnp.float32),
            pltpu.SemaphoreType.DMA,
            pltpu.SemaphoreType.DMA,
            pltpu.SemaphoreType.DMA,
        ],
    )


_PACK_R = 2000


def _pack_body(ent_ref, yf_ref, mf_ref, df_ref, yp_ref, mp_ref, dp_ref,
               ya_ref, ma_ref, da_ref, out_ref):
    z2 = jnp.zeros((_PACK_R, 2), jnp.float32)
    out_ref[...] = jnp.concatenate(
        [ent_ref[...], jnp.zeros((_PACK_R, 6), jnp.float32),
         yf_ref[...], mf_ref[...], df_ref[...], z2,
         yp_ref[...], mp_ref[...], dp_ref[...], z2,
         ya_ref[...], ma_ref[...], da_ref[...],
         jnp.zeros((_PACK_R, 66), jnp.float32)], axis=1)


def _pack_mega(entity_emb, year_freq, month_freq, day_freq,
               year_phi, month_phi, day_phi, year_amp, month_amp, day_amp):
    espec = pl.BlockSpec((_PACK_R, ENT_DIM), lambda i: (i, 0))
    tspec = pl.BlockSpec((_PACK_R, TIME_DIM), lambda i: (i, 0))
    return pl.pallas_call(
        _pack_body,
        grid=(NUM_ENT // _PACK_R,),
        in_specs=[espec] + [tspec] * 9,
        out_specs=pl.BlockSpec((_PACK_R, MEGA_W), lambda i: (i, 0)),
        out_shape=jax.ShapeDtypeStruct((NUM_ENT, MEGA_W), jnp.float32),
    )(entity_emb, year_freq, month_freq, day_freq,
      year_phi, month_phi, day_phi, year_amp, month_amp, day_amp)


_BLK = 512


def _tc_score_body(h_ref, t_ref, r_ref, y_ref, mo_ref, dy_ref, out_ref):
    y = y_ref[...]
    mo = mo_ref[...]
    dy = dy_ref[...]
    tvec = jnp.concatenate(
        [jnp.broadcast_to(y, (_BLK, TIME_DIM)),
         jnp.broadcast_to(mo, (_BLK, TIME_DIM)),
         jnp.broadcast_to(dy, (_BLK, TIME_DIM)),
         jnp.zeros((_BLK, 2), jnp.float32)], axis=1)

    h = h_ref[...]
    t = t_ref[...]
    r = r_ref[...]
    henc = h[:, 160:192] * jnp.sin(h[:, 96:128] * tvec + h[:, 128:160])
    tenc = t[:, 160:192] * jnp.sin(t[:, 96:128] * tvec + t[:, 128:160])
    denc = henc - tenc
    d10 = (denc[:, 0:10] + denc[:, 10:20] + denc[:, 20:30] + r[:, 96:106])
    d96 = h[:, :96] + r[:, :96] - t[:, :96]
    s = jnp.sum(d96 * d96, axis=1) + jnp.sum(d10 * d10, axis=1)
    out_ref[...] = jnp.sqrt(s)[:, None]


def _tc_score(h, t, r, y2, m2, d2, interpret=False):
    return pl.pallas_call(
        _tc_score_body,
        grid=(BATCH // _BLK,),
        in_specs=[
            pl.BlockSpec((_BLK, MEGA_W), lambda i: (i, 0)),
            pl.BlockSpec((_BLK, MEGA_W), lambda i: (i, 0)),
            pl.BlockSpec((_BLK, REL_W), lambda i: (i, 0)),
            pl.BlockSpec((_BLK, 1), lambda i: (i, 0)),
            pl.BlockSpec((_BLK, 1), lambda i: (i, 0)),
            pl.BlockSpec((_BLK, 1), lambda i: (i, 0)),
        ],
        out_specs=pl.BlockSpec((_BLK, 1), lambda i: (i, 0)),
        out_shape=jax.ShapeDtypeStruct((BATCH, 1), jnp.float32),
        interpret=interpret,
    )(h, t, r, y2, m2, d2)


def kernel(heads, rels, tails, years, months, days, entity_emb, relation_emb,
           year_freq, month_freq, day_freq, year_phi, month_phi, day_phi,
           year_amp, month_amp, day_amp):
    mega = _pack_mega(entity_emb, year_freq, month_freq, day_freq,
                      year_phi, month_phi, day_phi,
                      year_amp, month_amp, day_amp)
    zr = jnp.zeros((NUM_REL, 6), jnp.float32)
    relpad = jnp.concatenate(
        [relation_emb[:, :ENT_DIM], zr, relation_emb[:, ENT_DIM:],
         jnp.zeros((NUM_REL, 22), jnp.float32)], axis=1)
    hi = heads.astype(jnp.int32)
    ti = tails.astype(jnp.int32)
    ri = rels.astype(jnp.int32)
    h, t, r = _get_sc_gather()(hi, ti, ri, mega, relpad)
    y2 = years.reshape(BATCH, 1)
    m2 = months.reshape(BATCH, 1)
    d2 = days.reshape(BATCH, 1)
    scores = _tc_score(h, t, r, y2, m2, d2)
    return scores.reshape(-1)


# speedup vs baseline: 1.4564x; 1.0029x over previous
import functools
import jax
import jax.numpy as jnp
from jax import lax
from jax.experimental import pallas as pl
from jax.experimental.pallas import tpu as pltpu
from jax.experimental.pallas import tpu_sc as plsc

NUM_ENT = 100000
NUM_REL = 1000
ENT_DIM = 90
TIME_DIM = 10
BATCH = 16384
MEGA_W = 256
REL_W = 128

_NC, _NS = 2, 16
_NW = _NC * _NS
_BPW = BATCH // _NW
_CHUNK = 64
_NCHUNK = _BPW // _CHUNK


_RING = 3


def _sc_gather_body(heads_hbm, tails_hbm, rels_hbm, mega_hbm, rel_hbm,
                    h_out, t_out, r_out,
                    hidx, tidx, ridx,
                    hb0, hb1, hb2, tb0, tb1, tb2, rb0, rb1, rb2,
                    gsem, wsem0, wsem1, wsem2):
    wid = lax.axis_index("s") * _NC + lax.axis_index("c")
    base = wid * _BPW
    pltpu.sync_copy(heads_hbm.at[pl.ds(base, _BPW)], hidx)
    pltpu.sync_copy(tails_hbm.at[pl.ds(base, _BPW)], tidx)
    pltpu.sync_copy(rels_hbm.at[pl.ds(base, _BPW)], ridx)

    hb = (hb0, hb1, hb2)
    tb = (tb0, tb1, tb2)
    rb = (rb0, rb1, rb2)
    wsem = (wsem0, wsem1, wsem2)
    gd = [None] * _NCHUNK
    wd = [None] * _NCHUNK

    def issue_gather(ci):
        b = ci % _RING
        s = pl.ds(ci * _CHUNK, _CHUNK)
        gd[ci] = (
            pltpu.async_copy(mega_hbm.at[hidx.at[s]], hb[b], gsem),
            pltpu.async_copy(mega_hbm.at[tidx.at[s]], tb[b], gsem),
            pltpu.async_copy(rel_hbm.at[ridx.at[s]], rb[b], gsem),
        )

    def issue_write(ci):
        b = ci % _RING
        o = pl.ds(base + ci * _CHUNK, _CHUNK)
        wd[ci] = (
            pltpu.async_copy(hb[b], h_out.at[o], wsem[b]),
            pltpu.async_copy(tb[b], t_out.at[o], wsem[b]),
            pltpu.async_copy(rb[b], r_out.at[o], wsem[b]),
        )

    issue_gather(0)
    issue_gather(1)
    for ci in range(_NCHUNK):
        for d in gd[ci]:
            d.wait()
        issue_write(ci)
        if ci + 2 < _NCHUNK:
            if ci >= 1:
                for d in wd[ci - 1]:
                    d.wait()
            issue_gather(ci + 2)
    for ci in range(max(0, _NCHUNK - _RING), _NCHUNK):
        for d in wd[ci]:
            d.wait()


_sc_gather_cache = []


def _get_sc_gather():
    if not _sc_gather_cache:
        _sc_gather_cache.append(_build_sc_gather())
    return _sc_gather_cache[0]


def _build_sc_gather():
    return pl.kernel(
        _sc_gather_body,
        out_type=(
            jax.ShapeDtypeStruct((BATCH, MEGA_W), jnp.float32),
            jax.ShapeDtypeStruct((BATCH, MEGA_W), jnp.float32),
            jax.ShapeDtypeStruct((BATCH, REL_W), jnp.float32),
        ),
        mesh=plsc.VectorSubcoreMesh(core_axis_name="c", subcore_axis_name="s",
                                    num_cores=_NC, num_subcores=_NS),
        compiler_params=pltpu.CompilerParams(),
        scratch_types=[
            pltpu.VMEM((_BPW,), jnp.int32),
            pltpu.VMEM((_BPW,), jnp.int32),
            pltpu.VMEM((_BPW,), jnp.int32),
            pltpu.VMEM((_CHUNK, MEGA_W), jnp.float32),
            pltpu.VMEM((_CHUNK, MEGA_W), jnp.float32),
            pltpu.VMEM((_CHUNK, MEGA_W), jnp.float32),
            pltpu.VMEM((_CHUNK, MEGA_W), jnp.float32),
            pltpu.VMEM((_CHUNK, MEGA_W), jnp.float32),
            pltpu.VMEM((_CHUNK, MEGA_W), jnp.float32),
            pltpu.VMEM((_CHUNK, REL_W), jnp.float32),
            pltpu.VMEM((_CHUNK, REL_W), jnp.float32),
            pltpu.VMEM((_CHUNK, REL_W), jnp.float32),
            pltpu.SemaphoreType.DMA,
            pltpu.SemaphoreType.DMA,
            pltpu.SemaphoreType.DMA,
            pltpu.SemaphoreType.DMA,
        ],
    )


_PACK_R = 2000


def _pack_body(ent_ref, yf_ref, mf_ref, df_ref, yp_ref, mp_ref, dp_ref,
               ya_ref, ma_ref, da_ref, out_ref):
    z2 = jnp.zeros((_PACK_R, 2), jnp.float32)
    out_ref[...] = jnp.concatenate(
        [ent_ref[...], jnp.zeros((_PACK_R, 6), jnp.float32),
         yf_ref[...], mf_ref[...], df_ref[...], z2,
         yp_ref[...], mp_ref[...], dp_ref[...], z2,
         ya_ref[...], ma_ref[...], da_ref[...],
         jnp.zeros((_PACK_R, 66), jnp.float32)], axis=1)


def _pack_mega(entity_emb, year_freq, month_freq, day_freq,
               year_phi, month_phi, day_phi, year_amp, month_amp, day_amp):
    espec = pl.BlockSpec((_PACK_R, ENT_DIM), lambda i: (i, 0))
    tspec = pl.BlockSpec((_PACK_R, TIME_DIM), lambda i: (i, 0))
    return pl.pallas_call(
        _pack_body,
        grid=(NUM_ENT // _PACK_R,),
        in_specs=[espec] + [tspec] * 9,
        out_specs=pl.BlockSpec((_PACK_R, MEGA_W), lambda i: (i, 0)),
        out_shape=jax.ShapeDtypeStruct((NUM_ENT, MEGA_W), jnp.float32),
    )(entity_emb, year_freq, month_freq, day_freq,
      year_phi, month_phi, day_phi, year_amp, month_amp, day_amp)


_BLK = 512


def _tc_score_body(h_ref, t_ref, r_ref, y_ref, mo_ref, dy_ref, out_ref):
    y = y_ref[...]
    mo = mo_ref[...]
    dy = dy_ref[...]
    tvec = jnp.concatenate(
        [jnp.broadcast_to(y, (_BLK, TIME_DIM)),
         jnp.broadcast_to(mo, (_BLK, TIME_DIM)),
         jnp.broadcast_to(dy, (_BLK, TIME_DIM)),
         jnp.zeros((_BLK, 2), jnp.float32)], axis=1)

    h = h_ref[...]
    t = t_ref[...]
    r = r_ref[...]
    henc = h[:, 160:192] * jnp.sin(h[:, 96:128] * tvec + h[:, 128:160])
    tenc = t[:, 160:192] * jnp.sin(t[:, 96:128] * tvec + t[:, 128:160])
    denc = henc - tenc
    d10 = (denc[:, 0:10] + denc[:, 10:20] + denc[:, 20:30] + r[:, 96:106])
    d96 = h[:, :96] + r[:, :96] - t[:, :96]
    s = jnp.sum(d96 * d96, axis=1) + jnp.sum(d10 * d10, axis=1)
    out_ref[...] = jnp.sqrt(s)[:, None]


def _tc_score(h, t, r, y2, m2, d2, interpret=False):
    return pl.pallas_call(
        _tc_score_body,
        grid=(BATCH // _BLK,),
        in_specs=[
            pl.BlockSpec((_BLK, MEGA_W), lambda i: (i, 0)),
            pl.BlockSpec((_BLK, MEGA_W), lambda i: (i, 0)),
            pl.BlockSpec((_BLK, REL_W), lambda i: (i, 0)),
            pl.BlockSpec((_BLK, 1), lambda i: (i, 0)),
            pl.BlockSpec((_BLK, 1), lambda i: (i, 0)),
            pl.BlockSpec((_BLK, 1), lambda i: (i, 0)),
        ],
        out_specs=pl.BlockSpec((_BLK, 1), lambda i: (i, 0)),
        out_shape=jax.ShapeDtypeStruct((BATCH, 1), jnp.float32),
        interpret=interpret,
    )(h, t, r, y2, m2, d2)


def kernel(heads, rels, tails, years, months, days, entity_emb, relation_emb,
           year_freq, month_freq, day_freq, year_phi, month_phi, day_phi,
           year_amp, month_amp, day_amp):
    mega = _pack_mega(entity_emb, year_freq, month_freq, day_freq,
                      year_phi, month_phi, day_phi,
                      year_amp, month_amp, day_amp)
    zr = jnp.zeros((NUM_REL, 6), jnp.float32)
    relpad = jnp.concatenate(
        [relation_emb[:, :ENT_DIM], zr, relation_emb[:, ENT_DIM:],
         jnp.zeros((NUM_REL, 22), jnp.float32)], axis=1)
    hi = heads.astype(jnp.int32)
    ti = tails.astype(jnp.int32)
    ri = rels.astype(jnp.int32)
    h, t, r = _get_sc_gather()(hi, ti, ri, mega, relpad)
    y2 = years.reshape(BATCH, 1)
    m2 = months.reshape(BATCH, 1)
    d2 = days.reshape(BATCH, 1)
    scores = _tc_score(h, t, r, y2, m2, d2)
    return scores.reshape(-1)


# speedup vs baseline: 1.4626x; 1.0043x over previous
import functools
import jax
import jax.numpy as jnp
from jax import lax
from jax.experimental import pallas as pl
from jax.experimental.pallas import tpu as pltpu
from jax.experimental.pallas import tpu_sc as plsc

NUM_ENT = 100000
NUM_REL = 1000
ENT_DIM = 90
TIME_DIM = 10
BATCH = 16384
MEGA_W = 256
REL_W = 128

_NC, _NS = 2, 16
_NW = _NC * _NS
_BPW = BATCH // _NW
_CHUNK = 64
_NCHUNK = _BPW // _CHUNK


_RING = 3


def _sc_gather_body(heads_hbm, tails_hbm, rels_hbm, mega_hbm, rel_hbm,
                    h_out, t_out, r_out,
                    hidx, tidx, ridx,
                    hb0, hb1, hb2, tb0, tb1, tb2, rb0, rb1, rb2,
                    gsem, wsem0, wsem1, wsem2):
    wid = lax.axis_index("s") * _NC + lax.axis_index("c")
    base = wid * _BPW
    pltpu.sync_copy(heads_hbm.at[pl.ds(base, _BPW)], hidx)
    pltpu.sync_copy(tails_hbm.at[pl.ds(base, _BPW)], tidx)
    pltpu.sync_copy(rels_hbm.at[pl.ds(base, _BPW)], ridx)

    hb = (hb0, hb1, hb2)
    tb = (tb0, tb1, tb2)
    rb = (rb0, rb1, rb2)
    wsem = (wsem0, wsem1, wsem2)
    gd = [None] * _NCHUNK
    wd = [None] * _NCHUNK

    def issue_gather(ci):
        b = ci % _RING
        s = pl.ds(ci * _CHUNK, _CHUNK)
        gd[ci] = (
            pltpu.async_copy(mega_hbm.at[hidx.at[s]], hb[b], gsem),
            pltpu.async_copy(mega_hbm.at[tidx.at[s]], tb[b], gsem),
            pltpu.async_copy(rel_hbm.at[ridx.at[s]], rb[b], gsem),
        )

    def issue_write(ci):
        b = ci % _RING
        o = pl.ds(base + ci * _CHUNK, _CHUNK)
        wd[ci] = (
            pltpu.async_copy(hb[b], h_out.at[o], wsem[b]),
            pltpu.async_copy(tb[b], t_out.at[o], wsem[b]),
            pltpu.async_copy(rb[b], r_out.at[o], wsem[b]),
        )

    issue_gather(0)
    issue_gather(1)
    for ci in range(_NCHUNK):
        for d in gd[ci]:
            d.wait()
        issue_write(ci)
        if ci + 2 < _NCHUNK:
            if ci >= 1:
                for d in wd[ci - 1]:
                    d.wait()
            issue_gather(ci + 2)
    for ci in range(max(0, _NCHUNK - _RING), _NCHUNK):
        for d in wd[ci]:
            d.wait()


_sc_gather_cache = []


def _get_sc_gather():
    if not _sc_gather_cache:
        _sc_gather_cache.append(_build_sc_gather())
    return _sc_gather_cache[0]


def _build_sc_gather():
    return pl.kernel(
        _sc_gather_body,
        out_type=(
            jax.ShapeDtypeStruct((BATCH, MEGA_W), jnp.float32),
            jax.ShapeDtypeStruct((BATCH, MEGA_W), jnp.float32),
            jax.ShapeDtypeStruct((BATCH, REL_W), jnp.float32),
        ),
        mesh=plsc.VectorSubcoreMesh(core_axis_name="c", subcore_axis_name="s",
                                    num_cores=_NC, num_subcores=_NS),
        compiler_params=pltpu.CompilerParams(),
        scratch_types=[
            pltpu.VMEM((_BPW,), jnp.int32),
            pltpu.VMEM((_BPW,), jnp.int32),
            pltpu.VMEM((_BPW,), jnp.int32),
            pltpu.VMEM((_CHUNK, MEGA_W), jnp.float32),
            pltpu.VMEM((_CHUNK, MEGA_W), jnp.float32),
            pltpu.VMEM((_CHUNK, MEGA_W), jnp.float32),
            pltpu.VMEM((_CHUNK, MEGA_W), jnp.float32),
            pltpu.VMEM((_CHUNK, MEGA_W), jnp.float32),
            pltpu.VMEM((_CHUNK, MEGA_W), jnp.float32),
            pltpu.VMEM((_CHUNK, REL_W), jnp.float32),
            pltpu.VMEM((_CHUNK, REL_W), jnp.float32),
            pltpu.VMEM((_CHUNK, REL_W), jnp.float32),
            pltpu.SemaphoreType.DMA,
            pltpu.SemaphoreType.DMA,
            pltpu.SemaphoreType.DMA,
            pltpu.SemaphoreType.DMA,
        ],
    )


_PACK_R = 4000

import numpy as _np

_SEL_ENT = _np.zeros((ENT_DIM, MEGA_W), _np.float32)
for _j in range(ENT_DIM):
    _SEL_ENT[_j, _j] = 1.0
_SEL_T = []
for _k in range(9):
    _m = _np.zeros((TIME_DIM, MEGA_W), _np.float32)
    _col = 96 + 32 * (_k // 3) + TIME_DIM * (_k % 3)
    for _j in range(TIME_DIM):
        _m[_j, _col + _j] = 1.0
    _SEL_T.append(_m)


def _pack_body(ent_ref, yf_ref, mf_ref, df_ref, yp_ref, mp_ref, dp_ref,
               ya_ref, ma_ref, da_ref, sel_ent_ref, sel_t_ref, out_ref):
    acc = jnp.dot(ent_ref[...], sel_ent_ref[...],
                  preferred_element_type=jnp.float32)
    tcat = jnp.concatenate(
        [yf_ref[...], mf_ref[...], df_ref[...], yp_ref[...], mp_ref[...],
         dp_ref[...], ya_ref[...], ma_ref[...], da_ref[...]], axis=1)
    acc = acc + jnp.dot(tcat, sel_t_ref[...],
                        preferred_element_type=jnp.float32)
    out_ref[...] = acc


def _pack_mega(entity_emb, year_freq, month_freq, day_freq,
               year_phi, month_phi, day_phi, year_amp, month_amp, day_amp):
    espec = pl.BlockSpec((_PACK_R, ENT_DIM), lambda i: (i, 0))
    tspec = pl.BlockSpec((_PACK_R, TIME_DIM), lambda i: (i, 0))
    sel_ent = jnp.asarray(_SEL_ENT)
    sel_t = jnp.asarray(_np.concatenate(_SEL_T, axis=0))
    return pl.pallas_call(
        _pack_body,
        grid=(NUM_ENT // _PACK_R,),
        in_specs=[espec] + [tspec] * 9 + [
            pl.BlockSpec((ENT_DIM, MEGA_W), lambda i: (0, 0)),
            pl.BlockSpec((9 * TIME_DIM, MEGA_W), lambda i: (0, 0)),
        ],
        out_specs=pl.BlockSpec((_PACK_R, MEGA_W), lambda i: (i, 0)),
        out_shape=jax.ShapeDtypeStruct((NUM_ENT, MEGA_W), jnp.float32),
    )(entity_emb, year_freq, month_freq, day_freq,
      year_phi, month_phi, day_phi, year_amp, month_amp, day_amp,
      sel_ent, sel_t)


_BLK = 2048


def _tc_score_body(h_ref, t_ref, r_ref, y_ref, mo_ref, dy_ref, out_ref):
    y = y_ref[...]
    mo = mo_ref[...]
    dy = dy_ref[...]
    tvec = jnp.concatenate(
        [jnp.broadcast_to(y, (_BLK, TIME_DIM)),
         jnp.broadcast_to(mo, (_BLK, TIME_DIM)),
         jnp.broadcast_to(dy, (_BLK, TIME_DIM)),
         jnp.zeros((_BLK, 2), jnp.float32)], axis=1)

    h = h_ref[...]
    t = t_ref[...]
    r = r_ref[...]
    henc = h[:, 160:192] * jnp.sin(h[:, 96:128] * tvec + h[:, 128:160])
    tenc = t[:, 160:192] * jnp.sin(t[:, 96:128] * tvec + t[:, 128:160])
    denc = henc - tenc
    d10 = (denc[:, 0:10] + denc[:, 10:20] + denc[:, 20:30] + r[:, 96:106])
    d96 = h[:, :96] + r[:, :96] - t[:, :96]
    s = jnp.sum(d96 * d96, axis=1) + jnp.sum(d10 * d10, axis=1)
    out_ref[...] = jnp.sqrt(s)[:, None]


def _tc_score(h, t, r, y2, m2, d2, interpret=False):
    return pl.pallas_call(
        _tc_score_body,
        grid=(BATCH // _BLK,),
        in_specs=[
            pl.BlockSpec((_BLK, MEGA_W), lambda i: (i, 0)),
            pl.BlockSpec((_BLK, MEGA_W), lambda i: (i, 0)),
            pl.BlockSpec((_BLK, REL_W), lambda i: (i, 0)),
            pl.BlockSpec((_BLK, 1), lambda i: (i, 0)),
            pl.BlockSpec((_BLK, 1), lambda i: (i, 0)),
            pl.BlockSpec((_BLK, 1), lambda i: (i, 0)),
        ],
        out_specs=pl.BlockSpec((_BLK, 1), lambda i: (i, 0)),
        out_shape=jax.ShapeDtypeStruct((BATCH, 1), jnp.float32),
        interpret=interpret,
    )(h, t, r, y2, m2, d2)


def kernel(heads, rels, tails, years, months, days, entity_emb, relation_emb,
           year_freq, month_freq, day_freq, year_phi, month_phi, day_phi,
           year_amp, month_amp, day_amp):
    mega = _pack_mega(entity_emb, year_freq, month_freq, day_freq,
                      year_phi, month_phi, day_phi,
                      year_amp, month_amp, day_amp)
    zr = jnp.zeros((NUM_REL, 6), jnp.float32)
    relpad = jnp.concatenate(
        [relation_emb[:, :ENT_DIM], zr, relation_emb[:, ENT_DIM:],
         jnp.zeros((NUM_REL, 22), jnp.float32)], axis=1)
    hi = heads.astype(jnp.int32)
    ti = tails.astype(jnp.int32)
    ri = rels.astype(jnp.int32)
    h, t, r = _get_sc_gather()(hi, ti, ri, mega, relpad)
    y2 = years.reshape(BATCH, 1)
    m2 = months.reshape(BATCH, 1)
    d2 = days.reshape(BATCH, 1)
    scores = _tc_score(h, t, r, y2, m2, d2)
    return scores.reshape(-1)


# speedup vs baseline: 1.4739x; 1.0078x over previous
import functools
import jax
import jax.numpy as jnp
from jax import lax
from jax.experimental import pallas as pl
from jax.experimental.pallas import tpu as pltpu
from jax.experimental.pallas import tpu_sc as plsc

NUM_ENT = 100000
NUM_REL = 1000
ENT_DIM = 90
TIME_DIM = 10
BATCH = 16384
MEGA_W = 256
REL_W = 128

_NC, _NS = 2, 16
_NW = _NC * _NS
_BPW = BATCH // _NW
_CHUNK = 64
_NCHUNK = _BPW // _CHUNK


_RING = 3


def _sc_gather_body(heads_hbm, tails_hbm, mega_hbm,
                    h_out, t_out,
                    hidx, tidx,
                    hb0, hb1, hb2, tb0, tb1, tb2,
                    gsem, wsem0, wsem1, wsem2):
    wid = lax.axis_index("s") * _NC + lax.axis_index("c")
    base = wid * _BPW
    pltpu.sync_copy(heads_hbm.at[pl.ds(base, _BPW)], hidx)
    pltpu.sync_copy(tails_hbm.at[pl.ds(base, _BPW)], tidx)

    hb = (hb0, hb1, hb2)
    tb = (tb0, tb1, tb2)
    wsem = (wsem0, wsem1, wsem2)
    gd = [None] * _NCHUNK
    wd = [None] * _NCHUNK

    def issue_gather(ci):
        b = ci % _RING
        s = pl.ds(ci * _CHUNK, _CHUNK)
        gd[ci] = (
            pltpu.async_copy(mega_hbm.at[hidx.at[s]], hb[b], gsem),
            pltpu.async_copy(mega_hbm.at[tidx.at[s]], tb[b], gsem),
        )

    def issue_write(ci):
        b = ci % _RING
        o = pl.ds(base + ci * _CHUNK, _CHUNK)
        wd[ci] = (
            pltpu.async_copy(hb[b], h_out.at[o], wsem[b]),
            pltpu.async_copy(tb[b], t_out.at[o], wsem[b]),
        )

    issue_gather(0)
    issue_gather(1)
    for ci in range(_NCHUNK):
        for d in gd[ci]:
            d.wait()
        issue_write(ci)
        if ci + 2 < _NCHUNK:
            if ci >= 1:
                for d in wd[ci - 1]:
                    d.wait()
            issue_gather(ci + 2)
    for ci in range(max(0, _NCHUNK - _RING), _NCHUNK):
        for d in wd[ci]:
            d.wait()


_sc_gather_cache = []


def _get_sc_gather():
    if not _sc_gather_cache:
        _sc_gather_cache.append(_build_sc_gather())
    return _sc_gather_cache[0]


def _build_sc_gather():
    return pl.kernel(
        _sc_gather_body,
        out_type=(
            jax.ShapeDtypeStruct((BATCH, MEGA_W), jnp.float32),
            jax.ShapeDtypeStruct((BATCH, MEGA_W), jnp.float32),
        ),
        mesh=plsc.VectorSubcoreMesh(core_axis_name="c", subcore_axis_name="s",
                                    num_cores=_NC, num_subcores=_NS),
        compiler_params=pltpu.CompilerParams(),
        scratch_types=[
            pltpu.VMEM((_BPW,), jnp.int32),
            pltpu.VMEM((_BPW,), jnp.int32),
            pltpu.VMEM((_CHUNK, MEGA_W), jnp.float32),
            pltpu.VMEM((_CHUNK, MEGA_W), jnp.float32),
            pltpu.VMEM((_CHUNK, MEGA_W), jnp.float32),
            pltpu.VMEM((_CHUNK, MEGA_W), jnp.float32),
            pltpu.VMEM((_CHUNK, MEGA_W), jnp.float32),
            pltpu.VMEM((_CHUNK, MEGA_W), jnp.float32),
            pltpu.SemaphoreType.DMA,
            pltpu.SemaphoreType.DMA,
            pltpu.SemaphoreType.DMA,
            pltpu.SemaphoreType.DMA,
        ],
    )


_PACK_R = 4000

import numpy as _np

_SEL_ENT = _np.zeros((ENT_DIM, MEGA_W), _np.float32)
for _j in range(ENT_DIM):
    _SEL_ENT[_j, _j] = 1.0
_SEL_T = []
for _k in range(9):
    _m = _np.zeros((TIME_DIM, MEGA_W), _np.float32)
    _col = 96 + 32 * (_k // 3) + TIME_DIM * (_k % 3)
    for _j in range(TIME_DIM):
        _m[_j, _col + _j] = 1.0
    _SEL_T.append(_m)


def _pack_body(ent_ref, yf_ref, mf_ref, df_ref, yp_ref, mp_ref, dp_ref,
               ya_ref, ma_ref, da_ref, sel_ent_ref, sel_t_ref, out_ref):
    acc = jnp.dot(ent_ref[...], sel_ent_ref[...],
                  preferred_element_type=jnp.float32)
    tcat = jnp.concatenate(
        [yf_ref[...], mf_ref[...], df_ref[...], yp_ref[...], mp_ref[...],
         dp_ref[...], ya_ref[...], ma_ref[...], da_ref[...]], axis=1)
    acc = acc + jnp.dot(tcat, sel_t_ref[...],
                        preferred_element_type=jnp.float32)
    out_ref[...] = acc


def _pack_mega(entity_emb, year_freq, month_freq, day_freq,
               year_phi, month_phi, day_phi, year_amp, month_amp, day_amp):
    espec = pl.BlockSpec((_PACK_R, ENT_DIM), lambda i: (i, 0))
    tspec = pl.BlockSpec((_PACK_R, TIME_DIM), lambda i: (i, 0))
    sel_ent = jnp.asarray(_SEL_ENT)
    sel_t = jnp.asarray(_np.concatenate(_SEL_T, axis=0))
    return pl.pallas_call(
        _pack_body,
        grid=(NUM_ENT // _PACK_R,),
        in_specs=[espec] + [tspec] * 9 + [
            pl.BlockSpec((ENT_DIM, MEGA_W), lambda i: (0, 0)),
            pl.BlockSpec((9 * TIME_DIM, MEGA_W), lambda i: (0, 0)),
        ],
        out_specs=pl.BlockSpec((_PACK_R, MEGA_W), lambda i: (i, 0)),
        out_shape=jax.ShapeDtypeStruct((NUM_ENT, MEGA_W), jnp.float32),
    )(entity_emb, year_freq, month_freq, day_freq,
      year_phi, month_phi, day_phi, year_amp, month_amp, day_amp,
      sel_ent, sel_t)


_BLK = 2048


def _tc_score_body(h_ref, t_ref, rtab_ref, ri_ref, y_ref, mo_ref, dy_ref,
                   out_ref):
    ri = ri_ref[...]
    onehot = (ri == lax.broadcasted_iota(jnp.int32, (_BLK, NUM_REL), 1)
              ).astype(jnp.float32)
    r = jnp.dot(onehot, rtab_ref[...], preferred_element_type=jnp.float32)
    y = y_ref[...]
    mo = mo_ref[...]
    dy = dy_ref[...]
    tvec = jnp.concatenate(
        [jnp.broadcast_to(y, (_BLK, TIME_DIM)),
         jnp.broadcast_to(mo, (_BLK, TIME_DIM)),
         jnp.broadcast_to(dy, (_BLK, TIME_DIM)),
         jnp.zeros((_BLK, 2), jnp.float32)], axis=1)

    h = h_ref[...]
    t = t_ref[...]
    henc = h[:, 160:192] * jnp.sin(h[:, 96:128] * tvec + h[:, 128:160])
    tenc = t[:, 160:192] * jnp.sin(t[:, 96:128] * tvec + t[:, 128:160])
    denc = henc - tenc
    d10 = (denc[:, 0:10] + denc[:, 10:20] + denc[:, 20:30] + r[:, 96:106])
    d96 = h[:, :96] + r[:, :96] - t[:, :96]
    s = jnp.sum(d96 * d96, axis=1) + jnp.sum(d10 * d10, axis=1)
    out_ref[...] = jnp.sqrt(s)[:, None]


def _tc_score(h, t, rtab, ri2, y2, m2, d2, interpret=False):
    return pl.pallas_call(
        _tc_score_body,
        grid=(BATCH // _BLK,),
        in_specs=[
            pl.BlockSpec((_BLK, MEGA_W), lambda i: (i, 0)),
            pl.BlockSpec((_BLK, MEGA_W), lambda i: (i, 0)),
            pl.BlockSpec((NUM_REL, REL_W), lambda i: (0, 0)),
            pl.BlockSpec((_BLK, 1), lambda i: (i, 0)),
            pl.BlockSpec((_BLK, 1), lambda i: (i, 0)),
            pl.BlockSpec((_BLK, 1), lambda i: (i, 0)),
            pl.BlockSpec((_BLK, 1), lambda i: (i, 0)),
        ],
        out_specs=pl.BlockSpec((_BLK, 1), lambda i: (i, 0)),
        out_shape=jax.ShapeDtypeStruct((BATCH, 1), jnp.float32),
        interpret=interpret,
    )(h, t, rtab, ri2, y2, m2, d2)


def kernel(heads, rels, tails, years, months, days, entity_emb, relation_emb,
           year_freq, month_freq, day_freq, year_phi, month_phi, day_phi,
           year_amp, month_amp, day_amp):
    mega = _pack_mega(entity_emb, year_freq, month_freq, day_freq,
                      year_phi, month_phi, day_phi,
                      year_amp, month_amp, day_amp)
    zr = jnp.zeros((NUM_REL, 6), jnp.float32)
    relpad = jnp.concatenate(
        [relation_emb[:, :ENT_DIM], zr, relation_emb[:, ENT_DIM:],
         jnp.zeros((NUM_REL, 22), jnp.float32)], axis=1)
    hi = heads.astype(jnp.int32)
    ti = tails.astype(jnp.int32)
    ri2 = rels.astype(jnp.int32).reshape(BATCH, 1)
    h, t = _get_sc_gather()(hi, ti, mega)
    y2 = years.reshape(BATCH, 1)
    m2 = months.reshape(BATCH, 1)
    d2 = days.reshape(BATCH, 1)
    scores = _tc_score(h, t, relpad, ri2, y2, m2, d2)
    return scores.reshape(-1)
